# Initial kernel scaffold; baseline (speedup 1.0000x reference)
import jax
import jax.numpy as jnp
from jax.experimental import pallas as pl


def kernel(vertex, adj_distance, adj_angle, weights, bias):
    raise NotImplementedError("write your pallas kernel here")


# speedup vs baseline: 1.0010x; 1.0010x over previous
import functools

import jax
import jax.numpy as jnp
from jax.experimental import pallas as pl
from jax.experimental.pallas import tpu as pltpu


def _support_kernel(v_ref, w_ref, o_ref):
    o_ref[...] = jnp.dot(v_ref[...], w_ref[...],
                         preferred_element_type=jnp.float32)


def _gc_kernel(ad_ref, aa_ref, s_ref, b_ref, o_ref):
    a = ad_ref[...] + aa_ref[...]
    o_ref[...] = (jnp.dot(a, s_ref[...], preferred_element_type=jnp.float32)
                  + b_ref[...])


def kernel(vertex, adj_distance, adj_angle, weights, bias):
    n, in_f = vertex.shape
    out_f = weights.shape[1]

    support = pl.pallas_call(
        _support_kernel,
        out_shape=jax.ShapeDtypeStruct((n, out_f), jnp.float32),
    )(vertex, weights)

    tm = 80
    grid = (n // tm,)
    bias2 = bias.reshape(1, out_f)

    out = pl.pallas_call(
        _gc_kernel,
        grid=grid,
        in_specs=[
            pl.BlockSpec((tm, n), lambda m: (m, 0)),
            pl.BlockSpec((tm, n), lambda m: (m, 0)),
            pl.BlockSpec((n, out_f), lambda m: (0, 0)),
            pl.BlockSpec((1, out_f), lambda m: (0, 0)),
        ],
        out_specs=pl.BlockSpec((tm, out_f), lambda m: (m, 0)),
        out_shape=jax.ShapeDtypeStruct((n, out_f), jnp.float32),
        compiler_params=pltpu.CompilerParams(
            dimension_semantics=("arbitrary",),
        ),
    )(adj_distance, adj_angle, support, bias2)
    return out
